# Initial kernel scaffold; baseline (speedup 1.0000x reference)
import jax
import jax.numpy as jnp
from jax.experimental import pallas as pl


def kernel(xyz, points, W1, b1, g1, be1, W2, b2, g2, be2, W3, b3, g3, be3):
    raise NotImplementedError("write your pallas kernel here")


# speedup vs baseline: 10.8331x; 10.8331x over previous
import functools

import jax
import jax.numpy as jnp
from jax import lax
from jax.experimental import pallas as pl
from jax.experimental.pallas import tpu as pltpu
from jax.experimental.pallas import tpu_sc as plsc

B = 8
N = 2048
P = 512
K = 32
C0 = 128
NTOT = B * P * K
F32 = jnp.float32
I32 = jnp.int32


def _fps_body(xt_ref, f0_ref, nx_ref, ny_ref, nz_ref):
    x = xt_ref[0]
    y = xt_ref[1]
    z = xt_ref[2]
    iota_n = lax.broadcasted_iota(I32, (B, N), 1)
    iota_p = lax.broadcasted_iota(I32, (B, P), 1)

    def body(i, carry):
        distance, f, nx, ny, nz = carry
        mask = iota_n == f
        cx = jnp.sum(jnp.where(mask, x, 0.0), axis=1, keepdims=True)
        cy = jnp.sum(jnp.where(mask, y, 0.0), axis=1, keepdims=True)
        cz = jnp.sum(jnp.where(mask, z, 0.0), axis=1, keepdims=True)
        d = (x - cx) ** 2 + (y - cy) ** 2
        d = d + (z - cz) ** 2
        distance = jnp.minimum(distance, d)
        m = jnp.max(distance, axis=1, keepdims=True)
        fnew = jnp.min(jnp.where(distance == m, iota_n, N), axis=1, keepdims=True)
        sel = iota_p == i
        nx = jnp.where(sel, cx, nx)
        ny = jnp.where(sel, cy, ny)
        nz = jnp.where(sel, cz, nz)
        return (distance, fnew, nx, ny, nz)

    init = (
        jnp.full((B, N), 1e10, F32),
        f0_ref[...],
        jnp.zeros((B, P), F32),
        jnp.zeros((B, P), F32),
        jnp.zeros((B, P), F32),
    )
    _, _, nx, ny, nz = lax.fori_loop(0, P, body, init)
    nx_ref[...] = nx
    ny_ref[...] = ny
    nz_ref[...] = nz


def _fps(xyz_t, f0):
    return pl.pallas_call(
        _fps_body,
        out_shape=[jax.ShapeDtypeStruct((B, P), F32)] * 3,
    )(xyz_t, f0)


def _knn_body(xbt_ref, nx_ref, ny_ref, nz_ref, idx_ref):
    x = xbt_ref[0, 0:1, :]
    y = xbt_ref[0, 1:2, :]
    z = xbt_ref[0, 2:3, :]
    qx = nx_ref[0]
    qy = ny_ref[0]
    qz = nz_ref[0]
    dist = (qx - x) ** 2 + (qy - y) ** 2
    dist = dist + (qz - z) ** 2
    iota = lax.broadcasted_iota(I32, (P, N), 1)
    base = pl.program_id(0) * N
    inf = jnp.float32(jnp.inf)
    for j in range(K):
        m = jnp.min(dist, axis=1, keepdims=True)
        sel = jnp.min(jnp.where(dist == m, iota, N), axis=1, keepdims=True)
        idx_ref[0, :, j : j + 1] = sel + base
        dist = jnp.where(iota == sel, inf, dist)


def _knn(xyz_bt, nx3, ny3, nz3):
    return pl.pallas_call(
        _knn_body,
        grid=(B,),
        in_specs=[
            pl.BlockSpec((1, 3, N), lambda b: (b, 0, 0)),
            pl.BlockSpec((1, P, 1), lambda b: (b, 0, 0)),
            pl.BlockSpec((1, P, 1), lambda b: (b, 0, 0)),
            pl.BlockSpec((1, P, 1), lambda b: (b, 0, 0)),
        ],
        out_specs=pl.BlockSpec((1, P, K), lambda b: (b, 0, 0)),
        out_shape=jax.ShapeDtypeStruct((B, P, K), I32),
    )(xyz_bt, nx3, ny3, nz3)


def _table_body(m_ref, w_ref, t_ref):
    t_ref[...] = jnp.dot(m_ref[...], w_ref[...], preferred_element_type=F32)


def _table(m, wcat):
    rows = B * N
    tile = 2048
    return pl.pallas_call(
        _table_body,
        grid=(rows // tile,),
        in_specs=[
            pl.BlockSpec((tile, C0 + 3), lambda t: (t, 0)),
            pl.BlockSpec((C0 + 3, C0), lambda t: (0, 0)),
        ],
        out_specs=pl.BlockSpec((tile, C0), lambda t: (t, 0)),
        out_shape=jax.ShapeDtypeStruct((rows, C0), F32),
    )(m, wcat)


_NC = 2
_NS = 16
_NW = _NC * _NS
_CHUNK = 128
_PER_W = NTOT // _NW
_NCHUNK = _PER_W // _CHUNK


def _gather_sc_body(tab_ref, idx_ref, out_ref, idx_v, rows_v, sem):
    cid = lax.axis_index("c")
    sid = lax.axis_index("s")
    wid = sid * _NC + cid
    base = wid * _PER_W

    def body(j, _):
        off = base + j * _CHUNK
        pltpu.sync_copy(idx_ref.at[pl.ds(off, _CHUNK)], idx_v)
        pltpu.async_copy(tab_ref.at[idx_v], rows_v, sem).wait()
        pltpu.sync_copy(rows_v, out_ref.at[pl.ds(off, _CHUNK), :])
        return 0

    lax.fori_loop(0, _NCHUNK, body, 0)


def _gather_sc(table, idx_flat):
    mesh = plsc.VectorSubcoreMesh(
        core_axis_name="c", subcore_axis_name="s", num_cores=_NC, num_subcores=_NS
    )
    fn = pl.kernel(
        _gather_sc_body,
        out_type=jax.ShapeDtypeStruct((NTOT, C0), F32),
        mesh=mesh,
        scratch_types=[
            pltpu.VMEM((_CHUNK,), I32),
            pltpu.VMEM((_CHUNK, C0), F32),
            pltpu.SemaphoreType.DMA,
        ],
    )
    return fn(table, idx_flat)


QT = 64
GRID = (B * P) // QT
_EPS = 1e-5


def _y1_tile(g_ref, nxyz_ref, w1x_ref, b1_ref):
    bq = jnp.dot(nxyz_ref[...], w1x_ref[...], preferred_element_type=F32)
    return g_ref[...] - bq[:, None, :] + b1_ref[...][None]


def _bn_relu_tile(y, stats_ref, g_ref, be_ref, n):
    mean = stats_ref[0:1, :] * (1.0 / n)
    exx = stats_ref[1:2, :] * (1.0 / n)
    var = exx - mean * mean
    xh = (y - mean[None]) * lax.rsqrt(var[None] + _EPS)
    return jax.nn.relu(xh * g_ref[...][None] + be_ref[...][None])


def _accum_stats(y, acc, stats_ref):
    pid = pl.program_id(0)

    @pl.when(pid == 0)
    def _():
        acc[...] = jnp.zeros_like(acc)

    s = jnp.sum(y, axis=(0, 1))
    ss = jnp.sum(y * y, axis=(0, 1))
    acc[0:1, :] += s[None]
    acc[1:2, :] += ss[None]

    @pl.when(pid == GRID - 1)
    def _():
        stats_ref[...] = acc[...]


def _stats1_body(g_ref, nxyz_ref, w1x_ref, b1_ref, stats_ref, acc):
    y1 = _y1_tile(g_ref, nxyz_ref, w1x_ref, b1_ref)
    _accum_stats(y1, acc, stats_ref)


def _layer2_body(
    g_ref, nxyz_ref, w1x_ref, b1_ref, s1_ref, g1_ref, be1_ref, w2_ref, b2_ref,
    y2_ref, s2_ref, acc,
):
    y1 = _y1_tile(g_ref, nxyz_ref, w1x_ref, b1_ref)
    z1 = _bn_relu_tile(y1, s1_ref, g1_ref, be1_ref, NTOT)
    y2 = jnp.dot(
        z1.reshape(QT * K, C0), w2_ref[...], preferred_element_type=F32
    ) + b2_ref[...]
    y2 = y2.reshape(QT, K, C0)
    _accum_stats(y2, acc, s2_ref)
    y2_ref[...] = y2


def _layer3_body(y2_ref, s2_ref, g2_ref, be2_ref, w3_ref, b3_ref, y3_ref, s3_ref, acc):
    z2 = _bn_relu_tile(y2_ref[...], s2_ref, g2_ref, be2_ref, NTOT)
    y3 = jnp.dot(
        z2.reshape(QT * K, C0), w3_ref[...], preferred_element_type=F32
    ) + b3_ref[...]
    y3 = y3.reshape(QT, K, 2 * C0)
    _accum_stats(y3, acc, s3_ref)
    y3_ref[...] = y3


def _final_body(y3_ref, s3_ref, g3_ref, be3_ref, out_ref):
    z3 = _bn_relu_tile(y3_ref[...], s3_ref, g3_ref, be3_ref, NTOT)
    out_ref[...] = jnp.max(z3, axis=1)


def _tile_spec(c):
    return pl.BlockSpec((QT, K, c), lambda t: (t, 0, 0))


def _full_spec(r, c):
    return pl.BlockSpec((r, c), lambda t: (0, 0))


def _mlp(g3d, nxyz_flat, w1x, b1, g1, be1, w2, b2, g2, be2, w3, b3, g3, be3):
    nxyz_spec = pl.BlockSpec((QT, 3), lambda t: (t, 0))

    stats1 = pl.pallas_call(
        _stats1_body,
        grid=(GRID,),
        in_specs=[_tile_spec(C0), nxyz_spec, _full_spec(3, C0), _full_spec(1, C0)],
        out_specs=_full_spec(8, C0),
        out_shape=jax.ShapeDtypeStruct((8, C0), F32),
        scratch_shapes=[pltpu.VMEM((8, C0), F32)],
    )(g3d, nxyz_flat, w1x, b1)

    y2, stats2 = pl.pallas_call(
        _layer2_body,
        grid=(GRID,),
        in_specs=[
            _tile_spec(C0), nxyz_spec, _full_spec(3, C0), _full_spec(1, C0),
            _full_spec(8, C0), _full_spec(1, C0), _full_spec(1, C0),
            _full_spec(C0, C0), _full_spec(1, C0),
        ],
        out_specs=[_tile_spec(C0), _full_spec(8, C0)],
        out_shape=[
            jax.ShapeDtypeStruct((B * P, K, C0), F32),
            jax.ShapeDtypeStruct((8, C0), F32),
        ],
        scratch_shapes=[pltpu.VMEM((8, C0), F32)],
    )(g3d, nxyz_flat, w1x, b1, stats1, g1, be1, w2, b2)

    y3, stats3 = pl.pallas_call(
        _layer3_body,
        grid=(GRID,),
        in_specs=[
            _tile_spec(C0), _full_spec(8, C0), _full_spec(1, C0), _full_spec(1, C0),
            _full_spec(C0, 2 * C0), _full_spec(1, 2 * C0),
        ],
        out_specs=[_tile_spec(2 * C0), _full_spec(8, 2 * C0)],
        out_shape=[
            jax.ShapeDtypeStruct((B * P, K, 2 * C0), F32),
            jax.ShapeDtypeStruct((8, 2 * C0), F32),
        ],
        scratch_shapes=[pltpu.VMEM((8, 2 * C0), F32)],
    )(y2, stats2, g2, be2, w3, b3)

    out = pl.pallas_call(
        _final_body,
        grid=(GRID,),
        in_specs=[
            _tile_spec(2 * C0), _full_spec(8, 2 * C0),
            _full_spec(1, 2 * C0), _full_spec(1, 2 * C0),
        ],
        out_specs=pl.BlockSpec((QT, 2 * C0), lambda t: (t, 0)),
        out_shape=jax.ShapeDtypeStruct((B * P, 2 * C0), F32),
    )(y3, stats3, g3, be3)
    return out


@jax.jit
def kernel(xyz, points, W1, b1, g1, be1, W2, b2, g2, be2, W3, b3, g3, be3):
    f0 = jax.random.randint(jax.random.key(1), (B,), 0, N, dtype=I32).reshape(B, 1)
    xyz_t = jnp.transpose(xyz, (2, 0, 1))
    nx, ny, nz = _fps(xyz_t, f0)

    xyz_bt = jnp.transpose(xyz, (0, 2, 1))
    idxg = _knn(
        xyz_bt,
        nx.reshape(B, P, 1),
        ny.reshape(B, P, 1),
        nz.reshape(B, P, 1),
    )

    m = jnp.concatenate([points, xyz], axis=-1).reshape(B * N, C0 + 3)
    wcat = jnp.concatenate([W1[3:, :], W1[:3, :]], axis=0)
    table = _table(m, wcat)

    g3d = _gather_sc(table, idxg.reshape(NTOT)).reshape(B * P, K, C0)

    new_xyz = jnp.stack([nx, ny, nz], axis=-1)
    out = _mlp(
        g3d,
        new_xyz.reshape(B * P, 3),
        W1[:3, :],
        b1.reshape(1, C0), g1.reshape(1, C0), be1.reshape(1, C0),
        W2, b2.reshape(1, C0), g2.reshape(1, C0), be2.reshape(1, C0),
        W3, b3.reshape(1, 2 * C0), g3.reshape(1, 2 * C0), be3.reshape(1, 2 * C0),
    )
    return (new_xyz, out.reshape(B, P, 2 * C0))
